# baseline (speedup 1.0000x reference)
import functools

import jax
import jax.numpy as jnp
from jax import lax
from jax.experimental import pallas as pl
from jax.experimental.pallas import tpu as pltpu
from jax.experimental.pallas import tpu_sc as plsc

NWORDS = 1000000
NTAGS = 64
NTOK = 16384

NC = 2
NS = 16
LANES = 16
B_PER_SC = NTOK // NC
B_PER_W = B_PER_SC // NS
SCHUNK = 128
NSCHUNK = B_PER_W // SCHUNK

W_PAD = 1 << 20
W_PER_TILE = W_PAD // NS
ZBUF = 8192

_mesh = plsc.VectorSubcoreMesh(
    core_axis_name="c", subcore_axis_name="s", num_cores=NC, num_subcores=NS
)


def _hist_body(x_hbm, out_hbm, idx_v, ones_v, zbuf_v, acc_sh):
    cid = lax.axis_index("c")
    sid = lax.axis_index("s")

    def zfill(i, carry):
        zbuf_v[pl.ds(i * LANES, LANES)] = jnp.zeros((LANES,), jnp.float32)
        return carry

    lax.fori_loop(0, ZBUF // LANES, zfill, 0)
    for k in range(W_PER_TILE // ZBUF):
        pltpu.sync_copy(
            zbuf_v, acc_sh.at[pl.ds(sid * W_PER_TILE + k * ZBUF, ZBUF)]
        )

    def ofill(i, carry):
        ones_v[0, pl.ds(i * LANES, LANES)] = jnp.ones((LANES,), jnp.float32)
        return carry

    lax.fori_loop(0, SCHUNK // LANES, ofill, 0)

    base = cid * B_PER_SC + sid * B_PER_W
    for j in range(NSCHUNK):
        pltpu.sync_copy(x_hbm.at[pl.ds(base + j * SCHUNK, SCHUNK)], idx_v.at[j])
    plsc.subcore_barrier()
    for j in range(NSCHUNK):
        pltpu.sync_copy(ones_v.at[0], acc_sh.at[idx_v.at[j]], add=True)
    plsc.subcore_barrier()
    pltpu.sync_copy(
        acc_sh.at[pl.ds(sid * W_PER_TILE, W_PER_TILE)],
        out_hbm.at[cid, pl.ds(sid * W_PER_TILE, W_PER_TILE)],
    )


_hist = functools.partial(
    pl.kernel,
    mesh=_mesh,
    out_type=jax.ShapeDtypeStruct((NC, W_PAD), jnp.float32),
    scratch_types=[
        pltpu.VMEM((NSCHUNK, SCHUNK), jnp.int32),
        pltpu.VMEM((1, SCHUNK), jnp.float32),
        pltpu.VMEM((ZBUF,), jnp.float32),
        pltpu.VMEM_SHARED((W_PAD,), jnp.float32),
    ],
)(_hist_body)

BW = 32768
NB_TC = 22
SC_START = NB_TC * BW
PT = (W_PAD - SC_START) // (NC * NS)
SCW = 512
SCH = 256


def _matvec_tc_body(t_ref, c_ref, o_ref, acc_ref):
    i = pl.program_id(0)

    @pl.when(i == 0)
    def _init():
        acc_ref[...] = jnp.zeros_like(acc_ref)

    c = c_ref[0:1, :] + c_ref[1:2, :]
    acc_ref[...] += t_ref[...] * c

    @pl.when(i == NB_TC - 1)
    def _fin():
        o_ref[...] = jnp.sum(acc_ref[...], axis=1)[None, :]


def _matvec_sc_body(
    t_hbm, c_hbm, out_hbm, tbuf, tailbuf, c2_v, csum_v, acc_v, outv, s0, s1
):
    cid = lax.axis_index("c")
    sid = lax.axis_index("s")
    wid = cid * NS + sid
    start = SC_START + wid * PT
    width = jnp.maximum(jnp.minimum(start + PT, NWORDS) - start, 0)
    nfull = width // SCW

    for r in range(NC):
        pltpu.sync_copy(c_hbm.at[r, pl.ds(start, PT)], c2_v.at[r])

    def csfill(g, carry):
        sl = pl.ds(g * LANES, LANES)
        csum_v[sl] = c2_v[0, sl] + c2_v[1, sl]
        return carry

    lax.fori_loop(0, PT // LANES, csfill, 0)

    def accz(j, carry):
        acc_v[pl.ds(j * LANES, LANES)] = jnp.zeros((LANES,), jnp.float32)
        return carry

    lax.fori_loop(0, NTAGS, accz, 0)

    sems = (s0, s1)

    def issue(k, b):
        return pltpu.async_copy(
            t_hbm.at[:, pl.ds(start + k * SCW, SCW)], tbuf.at[b], sems[b]
        )

    def compute(k, b):
        tb = tbuf.at[b]
        for h in range(SCW // SCH):
            cs = [
                csum_v[pl.ds(k * SCW + h * SCH + g * LANES, LANES)]
                for g in range(SCH // LANES)
            ]

            def jbody(m, carry):
                for u in range(4):
                    j = m * 4 + u
                    sl = pl.ds(j * LANES, LANES)
                    a = acc_v[sl]
                    for g in range(SCH // LANES):
                        a = a + tb[j, pl.ds(h * SCH + g * LANES, LANES)] * cs[g]
                    acc_v[sl] = a
                return carry

            lax.fori_loop(0, NTAGS // 4, jbody, 0)

    @pl.when(nfull > 0)
    def _prologue():
        issue(0, 0)

    def pair(m, carry):
        issue(2 * m + 1, 1)
        pltpu.make_async_copy(
            t_hbm.at[:, pl.ds(start + 2 * m * SCW, SCW)], tbuf.at[0], s0
        ).wait()
        compute(2 * m, 0)

        @pl.when(2 * m + 2 < nfull)
        def _next():
            issue(2 * m + 2, 0)

        pltpu.make_async_copy(
            t_hbm.at[:, pl.ds(start + (2 * m + 1) * SCW, SCW)], tbuf.at[1], s1
        ).wait()
        compute(2 * m + 1, 1)
        return carry

    lax.fori_loop(0, nfull // 2, pair, 0)

    @pl.when(nfull % 2 == 1)
    def _odd():
        k = nfull - 1
        pltpu.make_async_copy(
            t_hbm.at[:, pl.ds(start + k * SCW, SCW)], tbuf.at[0], s0
        ).wait()
        compute(k, 0)

    rem = width - nfull * SCW

    @pl.when(rem > 0)
    def _tail():
        tail0 = start + nfull * SCW
        pltpu.sync_copy(t_hbm.at[:, pl.ds(tail0, 128)], tailbuf)
        cs = [csum_v[pl.ds(nfull * SCW + g * LANES, LANES)] for g in range(4)]

        def jbody(j, carry):
            sl = pl.ds(j * LANES, LANES)
            a = acc_v[sl]
            for g in range(4):
                a = a + tailbuf[j, pl.ds(g * LANES, LANES)] * cs[g]
            acc_v[sl] = a
            return carry

        lax.fori_loop(0, NTAGS, jbody, 0)

    def zout(i, carry):
        outv[i // 8, pl.ds((i % 8) * LANES, LANES)] = jnp.zeros(
            (LANES,), jnp.float32
        )
        return carry

    lax.fori_loop(0, LANES * 8, zout, 0)
    lane = lax.iota(jnp.int32, LANES)

    def red(j, carry):
        plsc.store_scatter(
            outv,
            [lane, jnp.full((LANES,), j, jnp.int32)],
            acc_v[pl.ds(j * LANES, LANES)],
        )
        return carry

    lax.fori_loop(0, NTAGS, red, 0)
    pltpu.sync_copy(outv, out_hbm.at[wid])


_matvec_sc = functools.partial(
    pl.kernel,
    mesh=_mesh,
    out_type=jax.ShapeDtypeStruct((NC * NS, LANES, 128), jnp.float32),
    scratch_types=[
        pltpu.VMEM((2, NTAGS, SCW), jnp.float32),
        pltpu.VMEM((NTAGS, 128), jnp.float32),
        pltpu.VMEM((NC, PT), jnp.float32),
        pltpu.VMEM((PT,), jnp.float32),
        pltpu.VMEM((NTAGS * LANES,), jnp.float32),
        pltpu.VMEM((LANES, 128), jnp.float32),
        pltpu.SemaphoreType.DMA,
        pltpu.SemaphoreType.DMA,
    ],
    compiler_params=pltpu.CompilerParams(needs_layout_passes=False),
)(_matvec_sc_body)


def _combine_body(tc_ref, sc_ref, b_ref, o_ref):
    s = jnp.sum(jnp.sum(sc_ref[...], axis=0), axis=0)[None, :]
    o_ref[...] = tc_ref[...] + s[:, :NTAGS] + b_ref[...]


def kernel(x, table, bias):
    counts = _hist(x.astype(jnp.int32))
    table_t = table.T
    tc_part = pl.pallas_call(
        _matvec_tc_body,
        grid=(NB_TC,),
        in_specs=[
            pl.BlockSpec((NTAGS, BW), lambda i: (0, i)),
            pl.BlockSpec((NC, BW), lambda i: (0, i)),
        ],
        out_specs=pl.BlockSpec((1, NTAGS), lambda i: (0, 0)),
        out_shape=jax.ShapeDtypeStruct((1, NTAGS), jnp.float32),
        scratch_shapes=[pltpu.VMEM((NTAGS, BW), jnp.float32)],
        compiler_params=pltpu.CompilerParams(
            dimension_semantics=("arbitrary",)
        ),
    )(table_t, counts)
    sc_part = _matvec_sc(table_t, counts)
    return pl.pallas_call(
        _combine_body,
        out_shape=jax.ShapeDtypeStruct((1, NTAGS), jnp.float32),
    )(tc_part, sc_part, bias.reshape(1, NTAGS))


# speedup vs baseline: 1.0391x; 1.0391x over previous
import functools

import jax
import jax.numpy as jnp
from jax import lax
from jax.experimental import pallas as pl
from jax.experimental.pallas import tpu as pltpu
from jax.experimental.pallas import tpu_sc as plsc

NWORDS = 1000000
NTAGS = 64
NTOK = 16384

NC = 2
NS = 16
LANES = 16
B_PER_SC = NTOK // NC
B_PER_W = B_PER_SC // NS
SCHUNK = 128
NSCHUNK = B_PER_W // SCHUNK

W_PAD = 1 << 20
W_PER_TILE = W_PAD // NS
ZBUF = 8192

_mesh = plsc.VectorSubcoreMesh(
    core_axis_name="c", subcore_axis_name="s", num_cores=NC, num_subcores=NS
)


def _hist_body(x_hbm, out_hbm, idx_v, ones_v, zbuf_v, acc_sh):
    cid = lax.axis_index("c")
    sid = lax.axis_index("s")

    def zfill(i, carry):
        zbuf_v[pl.ds(i * LANES, LANES)] = jnp.zeros((LANES,), jnp.float32)
        return carry

    lax.fori_loop(0, ZBUF // LANES, zfill, 0)
    for k in range(W_PER_TILE // ZBUF):
        pltpu.sync_copy(
            zbuf_v, acc_sh.at[pl.ds(sid * W_PER_TILE + k * ZBUF, ZBUF)]
        )

    def ofill(i, carry):
        ones_v[0, pl.ds(i * LANES, LANES)] = jnp.ones((LANES,), jnp.float32)
        return carry

    lax.fori_loop(0, SCHUNK // LANES, ofill, 0)

    base = cid * B_PER_SC + sid * B_PER_W
    for j in range(NSCHUNK):
        pltpu.sync_copy(x_hbm.at[pl.ds(base + j * SCHUNK, SCHUNK)], idx_v.at[j])
    plsc.subcore_barrier()
    for j in range(NSCHUNK):
        pltpu.sync_copy(ones_v.at[0], acc_sh.at[idx_v.at[j]], add=True)
    plsc.subcore_barrier()
    pltpu.sync_copy(
        acc_sh.at[pl.ds(sid * W_PER_TILE, W_PER_TILE)],
        out_hbm.at[cid, pl.ds(sid * W_PER_TILE, W_PER_TILE)],
    )


_hist = functools.partial(
    pl.kernel,
    mesh=_mesh,
    out_type=jax.ShapeDtypeStruct((NC, W_PAD), jnp.float32),
    scratch_types=[
        pltpu.VMEM((NSCHUNK, SCHUNK), jnp.int32),
        pltpu.VMEM((1, SCHUNK), jnp.float32),
        pltpu.VMEM((ZBUF,), jnp.float32),
        pltpu.VMEM_SHARED((W_PAD,), jnp.float32),
    ],
)(_hist_body)

BW = 32768
NB_TC = 23
SC_START = NB_TC * BW
PT = (W_PAD - SC_START) // (NC * NS)
SCW = 512
SCH = 256


def _matvec_tc_body(t_ref, c_ref, o_ref, acc_ref):
    i = pl.program_id(0)

    @pl.when(i == 0)
    def _init():
        acc_ref[...] = jnp.zeros_like(acc_ref)

    c = c_ref[0:1, :] + c_ref[1:2, :]
    acc_ref[...] += t_ref[...] * c

    @pl.when(i == NB_TC - 1)
    def _fin():
        o_ref[...] = jnp.sum(acc_ref[...], axis=1)[None, :]


def _matvec_sc_body(
    t_hbm, c_hbm, out_hbm, tbuf, tailbuf, c2_v, csum_v, acc_v, outv, s0, s1
):
    cid = lax.axis_index("c")
    sid = lax.axis_index("s")
    wid = cid * NS + sid
    start = SC_START + wid * PT
    width = jnp.maximum(jnp.minimum(start + PT, NWORDS) - start, 0)
    nfull = width // SCW

    for r in range(NC):
        pltpu.sync_copy(c_hbm.at[r, pl.ds(start, PT)], c2_v.at[r])

    def csfill(g, carry):
        sl = pl.ds(g * LANES, LANES)
        csum_v[sl] = c2_v[0, sl] + c2_v[1, sl]
        return carry

    lax.fori_loop(0, PT // LANES, csfill, 0)

    def accz(j, carry):
        acc_v[pl.ds(j * LANES, LANES)] = jnp.zeros((LANES,), jnp.float32)
        return carry

    lax.fori_loop(0, NTAGS, accz, 0)

    sems = (s0, s1)

    def issue(k, b):
        return pltpu.async_copy(
            t_hbm.at[:, pl.ds(start + k * SCW, SCW)], tbuf.at[b], sems[b]
        )

    def compute(k, b):
        tb = tbuf.at[b]
        for h in range(SCW // SCH):
            cs = [
                csum_v[pl.ds(k * SCW + h * SCH + g * LANES, LANES)]
                for g in range(SCH // LANES)
            ]

            def jbody(m, carry):
                for u in range(4):
                    j = m * 4 + u
                    sl = pl.ds(j * LANES, LANES)
                    a = acc_v[sl]
                    for g in range(SCH // LANES):
                        a = a + tb[j, pl.ds(h * SCH + g * LANES, LANES)] * cs[g]
                    acc_v[sl] = a
                return carry

            lax.fori_loop(0, NTAGS // 4, jbody, 0)

    @pl.when(nfull > 0)
    def _prologue():
        issue(0, 0)

    def pair(m, carry):
        issue(2 * m + 1, 1)
        pltpu.make_async_copy(
            t_hbm.at[:, pl.ds(start + 2 * m * SCW, SCW)], tbuf.at[0], s0
        ).wait()
        compute(2 * m, 0)

        @pl.when(2 * m + 2 < nfull)
        def _next():
            issue(2 * m + 2, 0)

        pltpu.make_async_copy(
            t_hbm.at[:, pl.ds(start + (2 * m + 1) * SCW, SCW)], tbuf.at[1], s1
        ).wait()
        compute(2 * m + 1, 1)
        return carry

    lax.fori_loop(0, nfull // 2, pair, 0)

    @pl.when(nfull % 2 == 1)
    def _odd():
        k = nfull - 1
        pltpu.make_async_copy(
            t_hbm.at[:, pl.ds(start + k * SCW, SCW)], tbuf.at[0], s0
        ).wait()
        compute(k, 0)

    rem = width - nfull * SCW

    @pl.when(rem > 0)
    def _tail():
        tail0 = start + nfull * SCW
        pltpu.sync_copy(t_hbm.at[:, pl.ds(tail0, 128)], tailbuf)
        cs = [csum_v[pl.ds(nfull * SCW + g * LANES, LANES)] for g in range(4)]

        def jbody(j, carry):
            sl = pl.ds(j * LANES, LANES)
            a = acc_v[sl]
            for g in range(4):
                a = a + tailbuf[j, pl.ds(g * LANES, LANES)] * cs[g]
            acc_v[sl] = a
            return carry

        lax.fori_loop(0, NTAGS, jbody, 0)

    def zout(i, carry):
        outv[i // 8, pl.ds((i % 8) * LANES, LANES)] = jnp.zeros(
            (LANES,), jnp.float32
        )
        return carry

    lax.fori_loop(0, LANES * 8, zout, 0)
    lane = lax.iota(jnp.int32, LANES)

    def red(j, carry):
        plsc.store_scatter(
            outv,
            [lane, jnp.full((LANES,), j, jnp.int32)],
            acc_v[pl.ds(j * LANES, LANES)],
        )
        return carry

    lax.fori_loop(0, NTAGS, red, 0)
    pltpu.sync_copy(outv, out_hbm.at[wid])


_matvec_sc = functools.partial(
    pl.kernel,
    mesh=_mesh,
    out_type=jax.ShapeDtypeStruct((NC * NS, LANES, 128), jnp.float32),
    scratch_types=[
        pltpu.VMEM((2, NTAGS, SCW), jnp.float32),
        pltpu.VMEM((NTAGS, 128), jnp.float32),
        pltpu.VMEM((NC, PT), jnp.float32),
        pltpu.VMEM((PT,), jnp.float32),
        pltpu.VMEM((NTAGS * LANES,), jnp.float32),
        pltpu.VMEM((LANES, 128), jnp.float32),
        pltpu.SemaphoreType.DMA,
        pltpu.SemaphoreType.DMA,
    ],
    compiler_params=pltpu.CompilerParams(needs_layout_passes=False),
)(_matvec_sc_body)


def _combine_body(tc_ref, sc_ref, b_ref, o_ref):
    s = jnp.sum(jnp.sum(sc_ref[...], axis=0), axis=0)[None, :]
    o_ref[...] = tc_ref[...] + s[:, :NTAGS] + b_ref[...]


def kernel(x, table, bias):
    counts = _hist(x.astype(jnp.int32))
    table_t = table.T
    tc_part = pl.pallas_call(
        _matvec_tc_body,
        grid=(NB_TC,),
        in_specs=[
            pl.BlockSpec((NTAGS, BW), lambda i: (0, i)),
            pl.BlockSpec((NC, BW), lambda i: (0, i)),
        ],
        out_specs=pl.BlockSpec((1, NTAGS), lambda i: (0, 0)),
        out_shape=jax.ShapeDtypeStruct((1, NTAGS), jnp.float32),
        scratch_shapes=[pltpu.VMEM((NTAGS, BW), jnp.float32)],
        compiler_params=pltpu.CompilerParams(
            dimension_semantics=("arbitrary",)
        ),
    )(table_t, counts)
    sc_part = _matvec_sc(table_t, counts)
    return pl.pallas_call(
        _combine_body,
        out_shape=jax.ShapeDtypeStruct((1, NTAGS), jnp.float32),
    )(tc_part, sc_part, bias.reshape(1, NTAGS))


# speedup vs baseline: 1.0418x; 1.0026x over previous
import functools

import jax
import jax.numpy as jnp
from jax import lax
from jax.experimental import pallas as pl
from jax.experimental.pallas import tpu as pltpu
from jax.experimental.pallas import tpu_sc as plsc

NWORDS = 1000000
NTAGS = 64
NTOK = 16384

NC = 2
NS = 16
LANES = 16
B_PER_SC = NTOK // NC
B_PER_W = B_PER_SC // NS
SCHUNK = 128
NSCHUNK = B_PER_W // SCHUNK

W_PAD = 1 << 20
W_PER_TILE = W_PAD // NS
ZBUF = 8192

_mesh = plsc.VectorSubcoreMesh(
    core_axis_name="c", subcore_axis_name="s", num_cores=NC, num_subcores=NS
)


def _hist_body(x_hbm, out_hbm, idx_v, ones_v, zbuf_v, acc_sh):
    cid = lax.axis_index("c")
    sid = lax.axis_index("s")

    def zfill(i, carry):
        zbuf_v[pl.ds(i * LANES, LANES)] = jnp.zeros((LANES,), jnp.float32)
        return carry

    lax.fori_loop(0, ZBUF // LANES, zfill, 0)
    for k in range(W_PER_TILE // ZBUF):
        pltpu.sync_copy(
            zbuf_v, acc_sh.at[pl.ds(sid * W_PER_TILE + k * ZBUF, ZBUF)]
        )

    def ofill(i, carry):
        ones_v[0, pl.ds(i * LANES, LANES)] = jnp.ones((LANES,), jnp.float32)
        return carry

    lax.fori_loop(0, SCHUNK // LANES, ofill, 0)

    base = cid * B_PER_SC + sid * B_PER_W
    for j in range(NSCHUNK):
        pltpu.sync_copy(x_hbm.at[pl.ds(base + j * SCHUNK, SCHUNK)], idx_v.at[j])
    plsc.subcore_barrier()
    for j in range(NSCHUNK):
        pltpu.sync_copy(ones_v.at[0], acc_sh.at[idx_v.at[j]], add=True)
    plsc.subcore_barrier()
    pltpu.sync_copy(
        acc_sh.at[pl.ds(sid * W_PER_TILE, W_PER_TILE)],
        out_hbm.at[cid, pl.ds(sid * W_PER_TILE, W_PER_TILE)],
    )


_hist = functools.partial(
    pl.kernel,
    mesh=_mesh,
    out_type=jax.ShapeDtypeStruct((NC, W_PAD), jnp.float32),
    scratch_types=[
        pltpu.VMEM((NSCHUNK, SCHUNK), jnp.int32),
        pltpu.VMEM((1, SCHUNK), jnp.float32),
        pltpu.VMEM((ZBUF,), jnp.float32),
        pltpu.VMEM_SHARED((W_PAD,), jnp.float32),
    ],
)(_hist_body)

BW = 32768
NB_TC = 26
SC_START = NB_TC * BW
PT = (W_PAD - SC_START) // (NC * NS)
SCW = 512
SCH = 256


def _matvec_tc_body(t_ref, c_ref, o_ref, acc_ref):
    i = pl.program_id(0)

    @pl.when(i == 0)
    def _init():
        acc_ref[...] = jnp.zeros_like(acc_ref)

    c = c_ref[0:1, :] + c_ref[1:2, :]
    acc_ref[...] += t_ref[...] * c

    @pl.when(i == NB_TC - 1)
    def _fin():
        o_ref[...] = jnp.sum(acc_ref[...], axis=1)[None, :]


def _matvec_sc_body(
    t_hbm, c_hbm, out_hbm, tbuf, tailbuf, c2_v, csum_v, acc_v, outv, s0, s1
):
    cid = lax.axis_index("c")
    sid = lax.axis_index("s")
    wid = cid * NS + sid
    start = SC_START + wid * PT
    width = jnp.maximum(jnp.minimum(start + PT, NWORDS) - start, 0)
    nfull = width // SCW

    for r in range(NC):
        pltpu.sync_copy(c_hbm.at[r, pl.ds(start, PT)], c2_v.at[r])

    def csfill(g, carry):
        sl = pl.ds(g * LANES, LANES)
        csum_v[sl] = c2_v[0, sl] + c2_v[1, sl]
        return carry

    lax.fori_loop(0, PT // LANES, csfill, 0)

    def accz(j, carry):
        acc_v[pl.ds(j * LANES, LANES)] = jnp.zeros((LANES,), jnp.float32)
        return carry

    lax.fori_loop(0, NTAGS, accz, 0)

    sems = (s0, s1)

    def issue(k, b):
        return pltpu.async_copy(
            t_hbm.at[:, pl.ds(start + k * SCW, SCW)], tbuf.at[b], sems[b]
        )

    def compute(k, b):
        tb = tbuf.at[b]
        for h in range(SCW // SCH):
            cs = [
                csum_v[pl.ds(k * SCW + h * SCH + g * LANES, LANES)]
                for g in range(SCH // LANES)
            ]

            def jbody(m, carry):
                for u in range(4):
                    j = m * 4 + u
                    sl = pl.ds(j * LANES, LANES)
                    a = acc_v[sl]
                    for g in range(SCH // LANES):
                        a = a + tb[j, pl.ds(h * SCH + g * LANES, LANES)] * cs[g]
                    acc_v[sl] = a
                return carry

            lax.fori_loop(0, NTAGS // 4, jbody, 0)

    @pl.when(nfull > 0)
    def _prologue():
        issue(0, 0)

    def pair(m, carry):
        issue(2 * m + 1, 1)
        pltpu.make_async_copy(
            t_hbm.at[:, pl.ds(start + 2 * m * SCW, SCW)], tbuf.at[0], s0
        ).wait()
        compute(2 * m, 0)

        @pl.when(2 * m + 2 < nfull)
        def _next():
            issue(2 * m + 2, 0)

        pltpu.make_async_copy(
            t_hbm.at[:, pl.ds(start + (2 * m + 1) * SCW, SCW)], tbuf.at[1], s1
        ).wait()
        compute(2 * m + 1, 1)
        return carry

    lax.fori_loop(0, nfull // 2, pair, 0)

    @pl.when(nfull % 2 == 1)
    def _odd():
        k = nfull - 1
        pltpu.make_async_copy(
            t_hbm.at[:, pl.ds(start + k * SCW, SCW)], tbuf.at[0], s0
        ).wait()
        compute(k, 0)

    rem = width - nfull * SCW

    @pl.when(rem > 0)
    def _tail():
        tail0 = start + nfull * SCW
        pltpu.sync_copy(t_hbm.at[:, pl.ds(tail0, 128)], tailbuf)
        cs = [csum_v[pl.ds(nfull * SCW + g * LANES, LANES)] for g in range(4)]

        def jbody(j, carry):
            sl = pl.ds(j * LANES, LANES)
            a = acc_v[sl]
            for g in range(4):
                a = a + tailbuf[j, pl.ds(g * LANES, LANES)] * cs[g]
            acc_v[sl] = a
            return carry

        lax.fori_loop(0, NTAGS, jbody, 0)

    def zout(i, carry):
        outv[i // 8, pl.ds((i % 8) * LANES, LANES)] = jnp.zeros(
            (LANES,), jnp.float32
        )
        return carry

    lax.fori_loop(0, LANES * 8, zout, 0)
    lane = lax.iota(jnp.int32, LANES)

    def red(j, carry):
        plsc.store_scatter(
            outv,
            [lane, jnp.full((LANES,), j, jnp.int32)],
            acc_v[pl.ds(j * LANES, LANES)],
        )
        return carry

    lax.fori_loop(0, NTAGS, red, 0)
    pltpu.sync_copy(outv, out_hbm.at[wid])


_matvec_sc = functools.partial(
    pl.kernel,
    mesh=_mesh,
    out_type=jax.ShapeDtypeStruct((NC * NS, LANES, 128), jnp.float32),
    scratch_types=[
        pltpu.VMEM((2, NTAGS, SCW), jnp.float32),
        pltpu.VMEM((NTAGS, 128), jnp.float32),
        pltpu.VMEM((NC, PT), jnp.float32),
        pltpu.VMEM((PT,), jnp.float32),
        pltpu.VMEM((NTAGS * LANES,), jnp.float32),
        pltpu.VMEM((LANES, 128), jnp.float32),
        pltpu.SemaphoreType.DMA,
        pltpu.SemaphoreType.DMA,
    ],
    compiler_params=pltpu.CompilerParams(needs_layout_passes=False),
)(_matvec_sc_body)


def _combine_body(tc_ref, sc_ref, b_ref, o_ref):
    s = jnp.sum(jnp.sum(sc_ref[...], axis=0), axis=0)[None, :]
    o_ref[...] = tc_ref[...] + s[:, :NTAGS] + b_ref[...]


def kernel(x, table, bias):
    counts = _hist(x.astype(jnp.int32))
    table_t = table.T
    tc_part = pl.pallas_call(
        _matvec_tc_body,
        grid=(NB_TC,),
        in_specs=[
            pl.BlockSpec((NTAGS, BW), lambda i: (0, i)),
            pl.BlockSpec((NC, BW), lambda i: (0, i)),
        ],
        out_specs=pl.BlockSpec((1, NTAGS), lambda i: (0, 0)),
        out_shape=jax.ShapeDtypeStruct((1, NTAGS), jnp.float32),
        scratch_shapes=[pltpu.VMEM((NTAGS, BW), jnp.float32)],
        compiler_params=pltpu.CompilerParams(
            dimension_semantics=("arbitrary",)
        ),
    )(table_t, counts)
    sc_part = _matvec_sc(table_t, counts)
    return pl.pallas_call(
        _combine_body,
        out_shape=jax.ShapeDtypeStruct((1, NTAGS), jnp.float32),
    )(tc_part, sc_part, bias.reshape(1, NTAGS))


# speedup vs baseline: 1.0625x; 1.0199x over previous
import functools

import jax
import jax.numpy as jnp
from jax import lax
from jax.experimental import pallas as pl
from jax.experimental.pallas import tpu as pltpu
from jax.experimental.pallas import tpu_sc as plsc

NWORDS = 1000000
NTAGS = 64
NTOK = 16384

NC = 2
NS = 16
LANES = 16
B_PER_SC = NTOK // NC
B_PER_W = B_PER_SC // NS
SCHUNK = 128
NSCHUNK = B_PER_W // SCHUNK

W_PAD = 1 << 20
W_PER_TILE = W_PAD // NS
ZBUF = 8192

_mesh = plsc.VectorSubcoreMesh(
    core_axis_name="c", subcore_axis_name="s", num_cores=NC, num_subcores=NS
)


def _hist_body(x_hbm, z_hbm, o1_hbm, out_hbm, idx_v, ones_v, zbuf_v, acc_sh):
    cid = lax.axis_index("c")
    sid = lax.axis_index("s")

    pltpu.sync_copy(z_hbm, zbuf_v)
    for k in range(W_PER_TILE // ZBUF):
        pltpu.sync_copy(
            zbuf_v, acc_sh.at[pl.ds(sid * W_PER_TILE + k * ZBUF, ZBUF)]
        )

    pltpu.sync_copy(o1_hbm, ones_v)

    base = cid * B_PER_SC + sid * B_PER_W
    for j in range(NSCHUNK):
        pltpu.sync_copy(x_hbm.at[pl.ds(base + j * SCHUNK, SCHUNK)], idx_v.at[j])
    plsc.subcore_barrier()
    for j in range(NSCHUNK):
        pltpu.sync_copy(ones_v, acc_sh.at[idx_v.at[j]], add=True)
    plsc.subcore_barrier()
    pltpu.sync_copy(
        acc_sh.at[pl.ds(sid * W_PER_TILE, W_PER_TILE)],
        out_hbm.at[pl.ds(cid * W_PAD + sid * W_PER_TILE, W_PER_TILE)],
    )


_hist = functools.partial(
    pl.kernel,
    mesh=_mesh,
    out_type=jax.ShapeDtypeStruct((NC * W_PAD,), jnp.float32),
    scratch_types=[
        pltpu.VMEM((NSCHUNK, SCHUNK), jnp.int32),
        pltpu.VMEM((SCHUNK,), jnp.float32),
        pltpu.VMEM((ZBUF,), jnp.float32),
        pltpu.VMEM_SHARED((W_PAD,), jnp.float32),
    ],
)(_hist_body)

BW = 32768
_GRID = pl.cdiv(NWORDS, BW)


def _matvec_body(t_ref, c0_ref, c1_ref, b_ref, o_ref, acc_ref):
    i = pl.program_id(0)

    @pl.when(i == 0)
    def _init():
        acc_ref[...] = jnp.zeros_like(acc_ref)

    c = (c0_ref[...] + c1_ref[...]).astype(jnp.float32)
    acc_ref[...] += t_ref[...] * c

    @pl.when(i == _GRID - 1)
    def _fin():
        o_ref[...] = jnp.sum(acc_ref[...], axis=1)[None, :] + b_ref[...]


def kernel(x, table, bias):
    counts = _hist(
        x.astype(jnp.int32),
        jnp.zeros((ZBUF,), jnp.float32),
        jnp.ones((SCHUNK,), jnp.float32),
    )
    counts2 = counts.reshape(1, NC * W_PAD)
    table_t = table.T
    return pl.pallas_call(
        _matvec_body,
        grid=(_GRID,),
        in_specs=[
            pl.BlockSpec((NTAGS, BW), lambda i: (0, i)),
            pl.BlockSpec((1, BW), lambda i: (0, i)),
            pl.BlockSpec((1, BW), lambda i: (0, W_PAD // BW + i)),
            pl.BlockSpec((1, NTAGS), lambda i: (0, 0)),
        ],
        out_specs=pl.BlockSpec((1, NTAGS), lambda i: (0, 0)),
        out_shape=jax.ShapeDtypeStruct((1, NTAGS), jnp.float32),
        scratch_shapes=[pltpu.VMEM((NTAGS, BW), jnp.float32)],
        compiler_params=pltpu.CompilerParams(
            dimension_semantics=("arbitrary",)
        ),
    )(table_t, counts2, counts2, bias.reshape(1, NTAGS))
